# baseline (speedup 1.0000x reference)
import functools

import jax
import jax.numpy as jnp
from jax import lax
from jax.experimental import pallas as pl
from jax.experimental.pallas import tpu as pltpu
from jax.experimental.pallas import tpu_sc as plsc

BATCH = 1024
SEQ = 200
EMBED_DIM = 64

NC, NS = 2, 16
NW = NC * NS
SENT_PER_W = BATCH // NW
S = 4
NGROUPS = SENT_PER_W // S
CHUNKS = ((0, 128), (128, 72))

_mesh = plsc.VectorSubcoreMesh(core_axis_name="c", subcore_axis_name="s")


@functools.partial(
    pl.kernel,
    out_type=jax.ShapeDtypeStruct((BATCH, SEQ, EMBED_DIM), jnp.float32),
    mesh=_mesh,
    compiler_params=pltpu.CompilerParams(use_tc_tiling_on_sc=False),
    scratch_types=[
        pltpu.VMEM((SENT_PER_W, SEQ), jnp.int32),
        pltpu.VMEM((2, S, SEQ, EMBED_DIM), jnp.float32),
        pltpu.SemaphoreType.DMA,
        pltpu.SemaphoreType.DMA,
    ],
)
def _gather_kernel(table_hbm, idx_hbm, out_hbm, idx_v, rows_v, gsem, osem):
    wid = lax.axis_index("s") * NC + lax.axis_index("c")
    sent_base = wid * SENT_PER_W

    pltpu.sync_copy(idx_hbm.at[pl.ds(sent_base, SENT_PER_W)], idx_v)

    def _fire(g):
        buf = rows_v.at[g % 2]
        for t in range(S):
            for (o, n) in CHUNKS:
                pltpu.async_copy(
                    table_hbm.at[idx_v.at[g * S + t, pl.ds(o, n)]],
                    buf.at[t, pl.ds(o, n)],
                    gsem,
                )

    def _out_slice(g):
        return out_hbm.at[pl.ds(sent_base + g * S, S)]

    _fire(0)

    @pl.loop(0, NGROUPS)
    def _group(g):
        @pl.when(g >= 1)
        def _():
            pltpu.make_async_copy(
                rows_v.at[(g + 1) % 2], _out_slice(g - 1), osem
            ).wait()

        @pl.when(g + 1 < NGROUPS)
        def _():
            _fire(g + 1)

        buf = rows_v.at[g % 2]
        for t in range(S):
            for (o, n) in CHUNKS:
                pltpu.make_async_copy(
                    table_hbm.at[idx_v.at[g * S + t, pl.ds(o, n)]],
                    buf.at[t, pl.ds(o, n)],
                    gsem,
                ).wait()
        pltpu.async_copy(buf, _out_slice(g), osem)

    pltpu.make_async_copy(
        rows_v.at[(NGROUPS - 1) % 2], _out_slice(NGROUPS - 1), osem
    ).wait()


def kernel(input_ids, table):
    return _gather_kernel(table, input_ids.astype(jnp.int32))


# speedup vs baseline: 1.0743x; 1.0743x over previous
import functools

import jax
import jax.numpy as jnp
from jax import lax
from jax.experimental import pallas as pl
from jax.experimental.pallas import tpu as pltpu
from jax.experimental.pallas import tpu_sc as plsc

BATCH = 1024
SEQ = 200
EMBED_DIM = 64
VOCAB = 100000

NC, NS = 2, 16
NW = NC * NS
NPASS = EMBED_DIM // NW
SC_CHUNK = 4
NCHUNK = SEQ // SC_CHUNK
LANES = 16

_mesh = plsc.VectorSubcoreMesh(core_axis_name="c", subcore_axis_name="s")


@functools.partial(
    pl.kernel,
    out_type=jax.ShapeDtypeStruct((SEQ, EMBED_DIM, BATCH), jnp.float32),
    mesh=_mesh,
    compiler_params=pltpu.CompilerParams(
        use_tc_tiling_on_sc=False, needs_layout_passes=False
    ),
    scratch_types=[
        pltpu.VMEM((VOCAB,), jnp.float32),
        pltpu.VMEM((2, SC_CHUNK, BATCH), jnp.int32),
        pltpu.VMEM((2, SC_CHUNK, 1, BATCH), jnp.float32),
        pltpu.SemaphoreType.DMA,
        pltpu.SemaphoreType.DMA,
        pltpu.SemaphoreType.DMA,
    ],
)
def _gather_kernel(tbl_hbm, ids_hbm, out_hbm, row_v, idx_v, out_v,
                   rsem, isem, osem):
    wid = lax.axis_index("s") * NC + lax.axis_index("c")

    def _ids_chunk(c):
        return ids_hbm.at[pl.ds(c * SC_CHUNK, SC_CHUNK)]

    for p in range(NPASS):
        d = p * NW + wid

        pltpu.async_copy(tbl_hbm.at[d], row_v, rsem)
        pltpu.async_copy(_ids_chunk(0), idx_v.at[0], isem)
        pltpu.make_async_copy(tbl_hbm.at[d], row_v, rsem).wait()

        def _out_slice(c):
            return out_hbm.at[pl.ds(c * SC_CHUNK, SC_CHUNK), pl.ds(d, 1)]

        @pl.loop(0, NCHUNK)
        def _chunk(c):
            pltpu.make_async_copy(_ids_chunk(c), idx_v.at[c % 2], isem).wait()

            @pl.when(c + 1 < NCHUNK)
            def _():
                pltpu.async_copy(
                    _ids_chunk(c + 1), idx_v.at[(c + 1) % 2], isem
                )

            @pl.when(c >= 2)
            def _():
                pltpu.make_async_copy(
                    out_v.at[c % 2], _out_slice(c - 2), osem
                ).wait()

            for s in range(SC_CHUNK):
                ib = idx_v.at[c % 2].at[s]
                ob = out_v.at[c % 2].at[s].at[0]

                @plsc.parallel_loop(0, BATCH // LANES, unroll=8)
                def _g(i):
                    iv = ib[pl.ds(i * LANES, LANES)]
                    ob[pl.ds(i * LANES, LANES)] = plsc.load_gather(
                        row_v, [iv]
                    )

            pltpu.async_copy(out_v.at[c % 2], _out_slice(c), osem)

        for cc in (NCHUNK - 2, NCHUNK - 1):
            pltpu.make_async_copy(
                out_v.at[cc % 2], _out_slice(cc), osem
            ).wait()


def kernel(input_ids, table):
    out_t = _gather_kernel(table.T, input_ids.T.astype(jnp.int32))
    return jnp.transpose(out_t, (2, 0, 1))


# speedup vs baseline: 2.2753x; 2.1178x over previous
import functools

import jax
import jax.numpy as jnp
from jax import lax
from jax.experimental import pallas as pl
from jax.experimental.pallas import tpu as pltpu
from jax.experimental.pallas import tpu_sc as plsc

BATCH = 1024
SEQ = 200
EMBED_DIM = 64
VOCAB = 100000

NC, NS = 2, 16
NW = NC * NS
NPASS = EMBED_DIM // NW
SC_CHUNK = 4
NCHUNK = SEQ // SC_CHUNK
LANES = 16
NBH = BATCH // 128

_mesh = plsc.VectorSubcoreMesh(core_axis_name="c", subcore_axis_name="s")


@functools.partial(
    pl.kernel,
    out_type=jax.ShapeDtypeStruct(
        (SEQ, EMBED_DIM // 8, NBH, 8, 128), jnp.float32
    ),
    mesh=_mesh,
    compiler_params=pltpu.CompilerParams(
        use_tc_tiling_on_sc=False, needs_layout_passes=False
    ),
    scratch_types=[
        pltpu.VMEM((VOCAB,), jnp.float32),
        pltpu.VMEM((2, SC_CHUNK, BATCH), jnp.int32),
        pltpu.VMEM((2, SC_CHUNK, 1, NBH, 1, 128), jnp.float32),
        pltpu.VMEM_SHARED((SEQ, BATCH), jnp.int32),
        pltpu.SemaphoreType.DMA,
        pltpu.SemaphoreType.DMA,
        pltpu.SemaphoreType.DMA,
    ],
)
def _gather_kernel(tbl_hbm, ids_hbm, out_hbm, row_v, idx_v, out_v, ids_sh,
                   rsem, isem, osem):
    cid = lax.axis_index("c")
    sid = lax.axis_index("s")
    wid = sid * NC + cid

    @pl.when(sid == 0)
    def _():
        pltpu.sync_copy(ids_hbm, ids_sh)

    plsc.subcore_barrier()

    def _ids_chunk(c):
        return ids_sh.at[pl.ds(c * SC_CHUNK, SC_CHUNK)]

    for p in range(NPASS):
        d = p * NW + wid
        dh = d // 8
        dl = d % 8

        pltpu.async_copy(tbl_hbm.at[d], row_v, rsem)
        pltpu.async_copy(_ids_chunk(0), idx_v.at[0], isem)
        pltpu.make_async_copy(tbl_hbm.at[d], row_v, rsem).wait()

        def _out_slice(c):
            return out_hbm.at[
                pl.ds(c * SC_CHUNK, SC_CHUNK),
                pl.ds(dh, 1),
                slice(None),
                pl.ds(dl, 1),
                slice(None),
            ]

        @pl.loop(0, NCHUNK)
        def _chunk(c):
            pltpu.make_async_copy(_ids_chunk(c), idx_v.at[c % 2], isem).wait()

            @pl.when(c + 1 < NCHUNK)
            def _():
                pltpu.async_copy(
                    _ids_chunk(c + 1), idx_v.at[(c + 1) % 2], isem
                )

            @pl.when(c >= 2)
            def _():
                pltpu.make_async_copy(
                    out_v.at[c % 2], _out_slice(c - 2), osem
                ).wait()

            for s in range(SC_CHUNK):
                ib = idx_v.at[c % 2].at[s]
                for bh in range(NBH):
                    ob = out_v.at[c % 2].at[s].at[0].at[bh].at[0]

                    @plsc.parallel_loop(0, 128 // LANES, unroll=8)
                    def _g(k):
                        iv = ib[pl.ds(bh * 128 + k * LANES, LANES)]
                        ob[pl.ds(k * LANES, LANES)] = plsc.load_gather(
                            row_v, [iv]
                        )

            pltpu.async_copy(out_v.at[c % 2], _out_slice(c), osem)

        for cc in (NCHUNK - 2, NCHUNK - 1):
            pltpu.make_async_copy(
                out_v.at[cc % 2], _out_slice(cc), osem
            ).wait()


def kernel(input_ids, table):
    phy = _gather_kernel(table.T, input_ids.T.astype(jnp.int32))
    out3 = jnp.transpose(phy, (0, 1, 3, 2, 4)).reshape(SEQ, EMBED_DIM, BATCH)
    return jnp.transpose(out3, (2, 0, 1))
